# baseline (speedup 1.0000x reference)
import jax
import jax.numpy as jnp
from jax import lax
from jax.experimental import pallas as pl
from jax.experimental.pallas import tpu as pltpu
from jax.experimental.pallas import tpu_sc as plsc

B = 16384
D = 64
NC = 2
NS = 16
NW = NC * NS
BPW = B // NW
CH = 128
NCH = BPW // CH


def _rsqrt(x):
    i = lax.bitcast_convert_type(x, jnp.int32)
    i = jnp.int32(0x5F3759DF) - (i >> 1)
    y = lax.bitcast_convert_type(i, jnp.float32)
    for _ in range(3):
        y = y * (1.5 - 0.5 * x * y * y)
    return y


def _tec_body(hid_hbm, lab_hbm, tid_hbm, ent_hbm, rel_hbm, out_hbm,
              hidx, lidx, tidx, hrows, trows, rrows, outv, *sems):
    wid = lax.axis_index("s") * NC + lax.axis_index("c")
    base = wid * BPW

    pltpu.sync_copy(hid_hbm.at[pl.ds(base, BPW)], hidx)
    pltpu.sync_copy(lab_hbm.at[pl.ds(base, BPW)], lidx)
    pltpu.sync_copy(tid_hbm.at[pl.ds(base, BPW)], tidx)

    lane = lax.broadcasted_iota(jnp.int32, (16,), 0)
    zero16 = jnp.zeros((16,), jnp.float32)
    bfly = [lane ^ k for k in (8, 4, 2, 1)]

    def hsum(x):
        for idx in bfly:
            x = x + x.at[idx].get(mode="promise_in_bounds", unique_indices=True)
        return x

    def fetch(ch, hrows, trows, rrows):
        cps = []
        for b in range(CH // 16):
            r0 = ch * CH + b * 16
            hv = hidx[pl.ds(r0, 16)]
            lv = lidx[pl.ds(r0, 16)]
            tv = tidx[pl.ds(r0, 16)]
            blk = []
            sem = sems[b]
            for j in range(16):
                row = b * 16 + j
                blk.append(pltpu.async_copy(ent_hbm.at[hv[j]], hrows.at[row], sem))
                blk.append(pltpu.async_copy(ent_hbm.at[tv[j]], trows.at[row], sem))
                blk.append(pltpu.async_copy(rel_hbm.at[lv[j]], rrows.at[row], sem))
            cps.append(blk)
        return cps

    def compute_block(ch, b, hrows, trows, rrows):
        acc = [zero16] * 6
        for j in range(16):
            i = b * 16 + j
            h = [hrows[i, pl.ds(16 * c, 16)] for c in range(4)]
            t = [trows[i, pl.ds(16 * c, 16)] for c in range(4)]
            r = [rrows[i, pl.ds(16 * c, 16)] for c in range(4)]
            prods = [
                sum(h[c] * h[c] for c in range(4)),
                sum(t[c] * t[c] for c in range(4)),
                sum(r[c] * r[c] for c in range(4)),
                sum(h[c] * r[c] for c in range(4)),
                sum(h[c] * t[c] for c in range(4)),
                sum(t[c] * r[c] for c in range(4)),
            ]
            m = lane == j
            acc = [jnp.where(m, hsum(p), a) for p, a in zip(prods, acc)]
        hh, tt, rr, hr, ht, tr = acc
        ih = _rsqrt(jnp.maximum(hh, 1e-24))
        it = _rsqrt(jnp.maximum(tt, 1e-24))
        ssd = (hh * ih * ih + rr + tt * it * it
               + 2.0 * (hr * ih) - 2.0 * (ht * (ih * it)) - 2.0 * (tr * it))
        ssd = jnp.maximum(ssd, 0.0)
        outv[pl.ds(ch * CH + b * 16, 16)] = ssd * _rsqrt(jnp.maximum(ssd, 1e-24))

    def chunk(ch, carry):
        cps = fetch(ch, hrows, trows, rrows)
        for b in range(CH // 16):
            for cp in cps[b]:
                cp.wait()
            compute_block(ch, b, hrows, trows, rrows)
        return carry

    lax.fori_loop(0, NCH, chunk, 0)

    pltpu.sync_copy(outv, out_hbm.at[pl.ds(base, BPW)])


@jax.jit
def _sc_transe(hid, lab, tid, ent_embs, rel_embs):
    mesh = plsc.VectorSubcoreMesh(core_axis_name="c", subcore_axis_name="s")
    f = pl.kernel(
        _tec_body,
        mesh=mesh,
        out_type=jax.ShapeDtypeStruct((B,), jnp.float32),
        scratch_types=[
            pltpu.VMEM((BPW,), jnp.int32),
            pltpu.VMEM((BPW,), jnp.int32),
            pltpu.VMEM((BPW,), jnp.int32),
            pltpu.VMEM((CH, D), jnp.float32),
            pltpu.VMEM((CH, D), jnp.float32),
            pltpu.VMEM((CH, D), jnp.float32),
            pltpu.VMEM((BPW,), jnp.float32),
        ] + [pltpu.SemaphoreType.DMA] * (CH // 16),
    )
    return f(hid, lab, tid, ent_embs, rel_embs)


def kernel(head_ind, label, tail_ind, ent_embs, rel_embs):
    hid = head_ind.astype(jnp.int32)
    lab = label.astype(jnp.int32)
    tid = tail_ind.astype(jnp.int32)
    return _sc_transe(hid, lab, tid, ent_embs, rel_embs)


# speedup vs baseline: 1.0033x; 1.0033x over previous
import jax
import jax.numpy as jnp
from jax import lax
from jax.experimental import pallas as pl
from jax.experimental.pallas import tpu as pltpu
from jax.experimental.pallas import tpu_sc as plsc

B = 16384
D = 64
NC = 2
NS = 16
NW = NC * NS
BPW = B // NW
CH = 128
NCH = BPW // CH


def _rsqrt(x):
    i = lax.bitcast_convert_type(x, jnp.int32)
    i = jnp.int32(0x5F3759DF) - (i >> 1)
    y = lax.bitcast_convert_type(i, jnp.float32)
    for _ in range(3):
        y = y * (1.5 - 0.5 * x * y * y)
    return y


def _tec_body(hid_hbm, lab_hbm, tid_hbm, ent_hbm, rel2_hbm, out_hbm,
              hidx, lidx, tidx, lidx2, lbits, hrows, trows, rrows2, outv,
              *sems):
    wid = lax.axis_index("s") * NC + lax.axis_index("c")
    base = wid * BPW

    pltpu.sync_copy(hid_hbm.at[pl.ds(base, BPW)], hidx)
    pltpu.sync_copy(lab_hbm.at[pl.ds(base, BPW)], lidx)
    pltpu.sync_copy(tid_hbm.at[pl.ds(base, BPW)], tidx)

    for g in range(BPW // 16):
        s = pl.ds(g * 16, 16)
        lv = lidx[s]
        lidx2[s] = lv >> 1
        lbits[s] = lv & 1

    lane = lax.broadcasted_iota(jnp.int32, (16,), 0)
    zero16 = jnp.zeros((16,), jnp.float32)
    bfly = [lane ^ k for k in (8, 4, 2, 1)]

    def hsum(x):
        for idx in bfly:
            x = x + x.at[idx].get(mode="promise_in_bounds", unique_indices=True)
        return x

    def fetch(ch, hrows, trows, rrows):
        cps = []
        rcp = pltpu.async_copy(
            rel2_hbm.at[lidx2.at[pl.ds(ch * CH, CH)]], rrows2, sems[-1])
        for b in range(CH // 16):
            r0 = ch * CH + b * 16
            hv = hidx[pl.ds(r0, 16)]
            tv = tidx[pl.ds(r0, 16)]
            blk = []
            sem = sems[b]
            for j in range(16):
                row = b * 16 + j
                blk.append(pltpu.async_copy(ent_hbm.at[hv[j]], hrows.at[row], sem))
                blk.append(pltpu.async_copy(ent_hbm.at[tv[j]], trows.at[row], sem))
            cps.append(blk)
        return cps, rcp

    def compute_block(ch, b, hrows, trows, rrows2):
        acc = [zero16] * 6
        bitv = lbits[pl.ds(ch * CH + b * 16, 16)]
        for j in range(16):
            i = b * 16 + j
            h = [hrows[i, pl.ds(16 * c, 16)] for c in range(4)]
            t = [trows[i, pl.ds(16 * c, 16)] for c in range(4)]
            rb = bitv[j] != 0
            rlo = [rrows2[i, pl.ds(16 * c, 16)] for c in range(4)]
            rhi = [rrows2[i, pl.ds(64 + 16 * c, 16)] for c in range(4)]
            r = [jnp.where(rb, a, o) for a, o in zip(rhi, rlo)]
            prods = [
                sum(h[c] * h[c] for c in range(4)),
                sum(t[c] * t[c] for c in range(4)),
                sum(r[c] * r[c] for c in range(4)),
                sum(h[c] * r[c] for c in range(4)),
                sum(h[c] * t[c] for c in range(4)),
                sum(t[c] * r[c] for c in range(4)),
            ]
            m = lane == j
            acc = [jnp.where(m, hsum(p), a) for p, a in zip(prods, acc)]
        hh, tt, rr, hr, ht, tr = acc
        ih = _rsqrt(jnp.maximum(hh, 1e-24))
        it = _rsqrt(jnp.maximum(tt, 1e-24))
        ssd = (hh * ih * ih + rr + tt * it * it
               + 2.0 * (hr * ih) - 2.0 * (ht * (ih * it)) - 2.0 * (tr * it))
        ssd = jnp.maximum(ssd, 0.0)
        outv[pl.ds(ch * CH + b * 16, 16)] = ssd * _rsqrt(jnp.maximum(ssd, 1e-24))

    def chunk(ch, carry):
        cps, rcp = fetch(ch, hrows, trows, rrows2)
        rcp.wait()
        for b in range(CH // 16):
            for cp in cps[b]:
                cp.wait()
            compute_block(ch, b, hrows, trows, rrows2)
        return carry

    lax.fori_loop(0, NCH, chunk, 0)

    pltpu.sync_copy(outv, out_hbm.at[pl.ds(base, BPW)])


@jax.jit
def _sc_transe(hid, lab, tid, ent_embs, rel_embs):
    mesh = plsc.VectorSubcoreMesh(core_axis_name="c", subcore_axis_name="s")
    f = pl.kernel(
        _tec_body,
        mesh=mesh,
        out_type=jax.ShapeDtypeStruct((B,), jnp.float32),
        scratch_types=[
            pltpu.VMEM((BPW,), jnp.int32),
            pltpu.VMEM((BPW,), jnp.int32),
            pltpu.VMEM((BPW,), jnp.int32),
            pltpu.VMEM((BPW,), jnp.int32),
            pltpu.VMEM((BPW,), jnp.int32),
            pltpu.VMEM((CH, D), jnp.float32),
            pltpu.VMEM((CH, D), jnp.float32),
            pltpu.VMEM((CH, 2 * D), jnp.float32),
            pltpu.VMEM((BPW,), jnp.float32),
        ] + [pltpu.SemaphoreType.DMA] * (CH // 16 + 1),
    )
    return f(hid, lab, tid, ent_embs, rel_embs)


def kernel(head_ind, label, tail_ind, ent_embs, rel_embs):
    hid = head_ind.astype(jnp.int32)
    lab = label.astype(jnp.int32)
    tid = tail_ind.astype(jnp.int32)
    rel2 = rel_embs.reshape(rel_embs.shape[0] // 2, 2 * D)
    return _sc_transe(hid, lab, tid, ent_embs, rel2)
